# Initial kernel scaffold; baseline (speedup 1.0000x reference)
import jax
import jax.numpy as jnp
from jax.experimental import pallas as pl


def kernel(x):
    raise NotImplementedError("write your pallas kernel here")


# speedup vs baseline: 1.2985x; 1.2985x over previous
import jax
import jax.numpy as jnp
from jax.experimental import pallas as pl
from jax.experimental.pallas import tpu as pltpu

_LOCAL_SIZE = 5
_ALPHA = 1e-4
_BETA = 0.75


def _lrn_body(x_ref, o_ref):
    x = x_ref[0]
    sq = x * x
    _, w = sq.shape
    z1 = jnp.zeros((1, w), sq.dtype)
    z2 = jnp.zeros((2, w), sq.dtype)
    acc = sq
    acc = acc + jnp.concatenate([sq[1:], z1], axis=0)
    acc = acc + jnp.concatenate([sq[2:], z2], axis=0)
    acc = acc + jnp.concatenate([z1, sq[:-1]], axis=0)
    acc = acc + jnp.concatenate([z2, sq[:-2]], axis=0)
    t = 1.0 + (_ALPHA / _LOCAL_SIZE) * acc
    o_ref[0] = x * jnp.exp(-_BETA * jnp.log(t))


def kernel(x):
    n, c, h, w = x.shape
    xf = x.reshape(n, c, h * w)
    out = pl.pallas_call(
        _lrn_body,
        out_shape=jax.ShapeDtypeStruct(xf.shape, xf.dtype),
        grid=(n,),
        in_specs=[pl.BlockSpec((1, c, h * w), lambda i: (i, 0, 0))],
        out_specs=pl.BlockSpec((1, c, h * w), lambda i: (i, 0, 0)),
        compiler_params=pltpu.CompilerParams(
            dimension_semantics=("parallel",),
        ),
        name="lrn_fused",
    )(xf)
    return out.reshape(n, c, h, w)
